# Initial kernel scaffold; baseline (speedup 1.0000x reference)
import jax
import jax.numpy as jnp
from jax.experimental import pallas as pl


def kernel(p, x, o, t1, W0, b0, Wd1, bd1, Wd2, bd2, Wd3, bd3, Wd4, bd4, Wu1, bu1, Wu2, bu2, Wu3, bu3, Wu4, bu4, W1, W2, b2, W3, b3):
    raise NotImplementedError("write your pallas kernel here")


# speedup vs baseline: 15.7549x; 15.7549x over previous
import jax
import jax.numpy as jnp
from jax import lax
from jax.experimental import pallas as pl
from jax.experimental.pallas import tpu as pltpu

_B, _N, _DIN, _K = 8, 1024, 32, 16
_BIG = 1e9


def _pdist2(pq, psT):
    d = None
    for c in range(3):
        diff = pq[:, c:c + 1] - psT[c:c + 1, :]
        sq = diff * diff
        d = sq if d is None else d + sq
    return d


def _argmin_onehot(D, iota):
    m = jnp.min(D, axis=1, keepdims=True)
    eq = D == m
    idx = jnp.min(jnp.where(eq, iota, _BIG), axis=1, keepdims=True)
    return (iota == idx).astype(jnp.float32), m


def _knn_max(D, Y, k):
    iota = lax.broadcasted_iota(jnp.int32, D.shape, 1).astype(jnp.float32)
    acc = None
    for _ in range(k):
        oh, _ = _argmin_onehot(D, iota)
        g = jnp.dot(oh, Y, preferred_element_type=jnp.float32)
        acc = g if acc is None else jnp.maximum(acc, g)
        D = D + oh * _BIG
    return acc


def _interp3(D, Ysrc):
    iota = lax.broadcasted_iota(jnp.int32, D.shape, 1).astype(jnp.float32)
    Wm = None
    for _ in range(3):
        oh, m = _argmin_onehot(D, iota)
        w = 1.0 / (jnp.maximum(m, 0.0) + 1e-8)
        t = oh * w
        Wm = t if Wm is None else Wm + t
        D = D + oh * _BIG
    Wm = Wm / jnp.sum(Wm, axis=1, keepdims=True)
    return jnp.dot(Wm, Ysrc, preferred_element_type=jnp.float32)


def _dot(a, b):
    return jnp.dot(a, b, preferred_element_type=jnp.float32)


def _body(x_ref, p0_ref, p0T_ref, p1_ref, p1T_ref, p2_ref, p2T_ref,
          p3_ref, p3T_ref,
          W0_ref, b0_ref,
          Wx1_ref, Wp1_ref, bd1_ref, Wx2_ref, Wp2_ref, bd2_ref,
          Wx3_ref, Wp3_ref, bd3_ref, Wx4_ref, Wp4_ref, bd4_ref,
          Wt1_ref, Wb1_ref, bu1_ref, Wt2_ref, Wb2_ref, bu2_ref,
          Wt3_ref, Wb3_ref, bu3_ref, Wt4_ref, Wb4_ref, bu4_ref,
          W1_ref, W2_ref, b2_ref, W3_ref, b3_ref,
          out_ref):
    x0 = x_ref[...]
    p0, p0T = p0_ref[0], p0T_ref[0]
    p1, p1T = p1_ref[0], p1T_ref[0]
    p2, p2T = p2_ref[0], p2T_ref[0]
    p3, p3T = p3_ref[0], p3T_ref[0]

    xb0 = jnp.maximum(_dot(x0, W0_ref[...]) + b0_ref[...], 0.0)

    def down(xb, p_src, pT_src, pq, Wx_ref, Wp_ref, b_ref):
        Wp = Wp_ref[...]
        Y = _dot(xb, Wx_ref[...]) + _dot(p_src, Wp)
        cq = b_ref[...] - _dot(pq, Wp)
        D = _pdist2(pq, pT_src)
        return jnp.maximum(_knn_max(D, Y, _K) + cq, 0.0)

    xb1 = down(xb0, p0, p0T, p1, Wx1_ref, Wp1_ref, bd1_ref)
    xb2 = down(xb1, p1, p1T, p2, Wx2_ref, Wp2_ref, bd2_ref)
    xb3 = down(xb2, p2, p2T, p3, Wx3_ref, Wp3_ref, bd3_ref)
    xb4 = down(xb3, p3, p3T, p3, Wx4_ref, Wp4_ref, bd4_ref)

    def up(x_src, pT_src, pq, xd, Wt_ref, Wb_ref, b_ref):
        D = _pdist2(pq, pT_src)
        xi = _interp3(D, x_src)
        h = _dot(xi, Wt_ref[...]) + _dot(xd, Wb_ref[...]) + b_ref[...]
        return jnp.maximum(h, 0.0)

    xu1 = up(xb4, p3T, p3, xb3, Wt1_ref, Wb1_ref, bu1_ref)
    xu2 = up(xu1, p3T, p2, xb2, Wt2_ref, Wb2_ref, bu2_ref)
    xu3 = up(xu2, p2T, p1, xb1, Wt3_ref, Wb3_ref, bu3_ref)
    xu4 = up(xu3, p1T, p0, xb0, Wt4_ref, Wb4_ref, bu4_ref)

    h = jnp.maximum(_dot(xu4, W1_ref[...]), 0.0)
    h = jnp.maximum(_dot(h, W2_ref[...]) + b2_ref[...], 0.0)
    out_ref[...] = jnp.tanh(_dot(h, W3_ref[...]) + b3_ref[...])


def kernel(p, x, o, t1, W0, b0, Wd1, bd1, Wd2, bd2, Wd3, bd3, Wd4, bd4,
           Wu1, bu1, Wu2, bu2, Wu3, bu3, Wu4, bu4, W1, W2, b2, W3, b3):
    del o, t1
    f32 = jnp.float32

    p0 = p
    p1 = p0[:, ::4, :]
    p2 = p1[:, ::3, :]
    p3 = p2[:, ::2, :]
    pTs = [jnp.swapaxes(q, 1, 2) for q in (p0, p1, p2, p3)]

    enc = []
    for W, b, din in ((Wd1, bd1, 32), (Wd2, bd2, 48), (Wd3, bd3, 72),
                      (Wd4, bd4, 108)):
        enc += [W[:din], W[din:din + 3], b.reshape(1, -1)]
    dec = []
    for W, b, dsrc in ((Wu1, bu1, 162), (Wu2, bu2, 108), (Wu3, bu3, 72),
                       (Wu4, bu4, 48)):
        dec += [W[:dsrc], W[dsrc:], b.reshape(1, -1)]

    n1, n2, n3 = p1.shape[1], p2.shape[1], p3.shape[1]

    def pspec(n):
        return pl.BlockSpec((1, n, 3), lambda bi: (bi, 0, 0))

    def pTspec(n):
        return pl.BlockSpec((1, 3, n), lambda bi: (bi, 0, 0))

    def wspec(a):
        return pl.BlockSpec(a.shape, lambda bi: (0,) * a.ndim)

    weights = [W0, b0.reshape(1, -1)] + enc + dec + [
        W1, W2, b2.reshape(1, -1), W3, b3.reshape(1, -1)]

    out = pl.pallas_call(
        _body,
        grid=(_B,),
        in_specs=[pl.BlockSpec((_N, _DIN), lambda bi: (bi, 0)),
                  pspec(_N), pTspec(_N), pspec(n1), pTspec(n1),
                  pspec(n2), pTspec(n2), pspec(n3), pTspec(n3)]
                 + [wspec(w) for w in weights],
        out_specs=pl.BlockSpec((_N, 3), lambda bi: (bi, 0)),
        out_shape=jax.ShapeDtypeStruct((_B * _N, 3), f32),
        compiler_params=pltpu.CompilerParams(
            dimension_semantics=("arbitrary",)),
    )(x, p0, pTs[0], p1, pTs[1], p2, pTs[2], p3, pTs[3], *weights)
    return out.reshape(_B, _N, 3)


# speedup vs baseline: 16.5794x; 1.0523x over previous
import jax
import jax.numpy as jnp
from jax import lax
from jax.experimental import pallas as pl
from jax.experimental.pallas import tpu as pltpu

_B, _N, _DIN, _K = 8, 1024, 32, 16
_PAIR = 1
_BIG = 1e9


def _pdist2(pq, psT):
    d = None
    for c in range(3):
        diff = pq[:, c:c + 1] - psT[c:c + 1, :]
        sq = diff * diff
        d = sq if d is None else d + sq
    return d


def _argmin_onehot(D, iota):
    m = jnp.min(D, axis=1, keepdims=True)
    eq = D == m
    idx = jnp.min(jnp.where(eq, iota, _BIG), axis=1, keepdims=True)
    return iota == idx, m


def _knn_max(D, Y, k):
    iota = lax.broadcasted_iota(jnp.int32, D.shape, 1).astype(jnp.float32)
    acc = None
    for _ in range(k):
        ohb, _ = _argmin_onehot(D, iota)
        g = jnp.dot(ohb.astype(jnp.float32), Y,
                    preferred_element_type=jnp.float32)
        acc = g if acc is None else jnp.maximum(acc, g)
        D = jnp.where(ohb, _BIG, D)
    return acc


def _interp3(D, Ysrc):
    iota = lax.broadcasted_iota(jnp.int32, D.shape, 1).astype(jnp.float32)
    Wm = None
    for _ in range(3):
        ohb, m = _argmin_onehot(D, iota)
        w = 1.0 / (jnp.maximum(m, 0.0) + 1e-8)
        t = jnp.where(ohb, w, 0.0)
        Wm = t if Wm is None else Wm + t
        D = jnp.where(ohb, _BIG, D)
    Wm = Wm / jnp.sum(Wm, axis=1, keepdims=True)
    return jnp.dot(Wm, Ysrc, preferred_element_type=jnp.float32)


def _dot(a, b):
    return jnp.dot(a, b, preferred_element_type=jnp.float32)


def _network(x0, ps, pTs, wr):
    (W0_ref, b0_ref,
     Wx1_ref, Wp1_ref, bd1_ref, Wx2_ref, Wp2_ref, bd2_ref,
     Wx3_ref, Wp3_ref, bd3_ref, Wx4_ref, Wp4_ref, bd4_ref,
     Wt1_ref, Wb1_ref, bu1_ref, Wt2_ref, Wb2_ref, bu2_ref,
     Wt3_ref, Wb3_ref, bu3_ref, Wt4_ref, Wb4_ref, bu4_ref,
     W1_ref, W2_ref, b2_ref, W3_ref, b3_ref) = wr
    p0, p1, p2, p3 = ps
    p0T, p1T, p2T, p3T = pTs

    xb0 = jnp.maximum(_dot(x0, W0_ref[...]) + b0_ref[...], 0.0)

    def down(xb, p_src, pT_src, pq, Wx_ref, Wp_ref, b_ref):
        Wp = Wp_ref[...]
        Y = _dot(xb, Wx_ref[...]) + _dot(p_src, Wp)
        cq = b_ref[...] - _dot(pq, Wp)
        D = _pdist2(pq, pT_src)
        return jnp.maximum(_knn_max(D, Y, _K) + cq, 0.0)

    xb1 = down(xb0, p0, p0T, p1, Wx1_ref, Wp1_ref, bd1_ref)
    xb2 = down(xb1, p1, p1T, p2, Wx2_ref, Wp2_ref, bd2_ref)
    xb3 = down(xb2, p2, p2T, p3, Wx3_ref, Wp3_ref, bd3_ref)
    xb4 = down(xb3, p3, p3T, p3, Wx4_ref, Wp4_ref, bd4_ref)

    def up(x_src, pT_src, pq, xd, Wt_ref, Wb_ref, b_ref):
        D = _pdist2(pq, pT_src)
        xi = _interp3(D, x_src)
        h = _dot(xi, Wt_ref[...]) + _dot(xd, Wb_ref[...]) + b_ref[...]
        return jnp.maximum(h, 0.0)

    xu1 = up(xb4, p3T, p3, xb3, Wt1_ref, Wb1_ref, bu1_ref)
    xu2 = up(xu1, p3T, p2, xb2, Wt2_ref, Wb2_ref, bu2_ref)
    xu3 = up(xu2, p2T, p1, xb1, Wt3_ref, Wb3_ref, bu3_ref)
    xu4 = up(xu3, p1T, p0, xb0, Wt4_ref, Wb4_ref, bu4_ref)

    h = jnp.maximum(_dot(xu4, W1_ref[...]), 0.0)
    h = jnp.maximum(_dot(h, W2_ref[...]) + b2_ref[...], 0.0)
    return jnp.tanh(_dot(h, W3_ref[...]) + b3_ref[...])


def _body(x_ref, p0_ref, p0T_ref, p1_ref, p1T_ref, p2_ref, p2T_ref,
          p3_ref, p3T_ref, *wr_and_out):
    wr, out_ref = wr_and_out[:-1], wr_and_out[-1]
    for i in range(_PAIR):
        ps = [p0_ref[i], p1_ref[i], p2_ref[i], p3_ref[i]]
        pTs = [p0T_ref[i], p1T_ref[i], p2T_ref[i], p3T_ref[i]]
        x0 = x_ref[pl.ds(i * _N, _N), :]
        out_ref[pl.ds(i * _N, _N), :] = _network(x0, ps, pTs, wr)


def kernel(p, x, o, t1, W0, b0, Wd1, bd1, Wd2, bd2, Wd3, bd3, Wd4, bd4,
           Wu1, bu1, Wu2, bu2, Wu3, bu3, Wu4, bu4, W1, W2, b2, W3, b3):
    del o, t1
    f32 = jnp.float32

    p0 = p
    p1 = p0[:, ::4, :]
    p2 = p1[:, ::3, :]
    p3 = p2[:, ::2, :]
    pTs = [jnp.swapaxes(q, 1, 2) for q in (p0, p1, p2, p3)]

    enc = []
    for W, b, din in ((Wd1, bd1, 32), (Wd2, bd2, 48), (Wd3, bd3, 72),
                      (Wd4, bd4, 108)):
        enc += [W[:din], W[din:din + 3], b.reshape(1, -1)]
    dec = []
    for W, b, dsrc in ((Wu1, bu1, 162), (Wu2, bu2, 108), (Wu3, bu3, 72),
                       (Wu4, bu4, 48)):
        dec += [W[:dsrc], W[dsrc:], b.reshape(1, -1)]

    n1, n2, n3 = p1.shape[1], p2.shape[1], p3.shape[1]

    def pspec(n):
        return pl.BlockSpec((_PAIR, n, 3), lambda bi: (bi, 0, 0))

    def pTspec(n):
        return pl.BlockSpec((_PAIR, 3, n), lambda bi: (bi, 0, 0))

    def wspec(a):
        return pl.BlockSpec(a.shape, lambda bi: (0,) * a.ndim)

    weights = [W0, b0.reshape(1, -1)] + enc + dec + [
        W1, W2, b2.reshape(1, -1), W3, b3.reshape(1, -1)]

    out = pl.pallas_call(
        _body,
        grid=(_B // _PAIR,),
        in_specs=[pl.BlockSpec((_PAIR * _N, _DIN), lambda bi: (bi, 0)),
                  pspec(_N), pTspec(_N), pspec(n1), pTspec(n1),
                  pspec(n2), pTspec(n2), pspec(n3), pTspec(n3)]
                 + [wspec(w) for w in weights],
        out_specs=pl.BlockSpec((_PAIR * _N, 3), lambda bi: (bi, 0)),
        out_shape=jax.ShapeDtypeStruct((_B * _N, 3), f32),
        compiler_params=pltpu.CompilerParams(
            dimension_semantics=("arbitrary",)),
    )(x, p0, pTs[0], p1, pTs[1], p2, pTs[2], p3, pTs[3], *weights)
    return out.reshape(_B, _N, 3)
